# baseline (speedup 1.0000x reference)
import jax
import jax.numpy as jnp
from jax import lax
from jax.experimental import pallas as pl
from jax.experimental.pallas import tpu as pltpu
from jax.experimental.pallas import tpu_sc as plsc

NC = 2
NS = 16
NW = NC * NS
LANES = 16

N = 10000
D = 128
E = 320000

N_PAD = 10240
SR = N_PAD // NS
EPT = E // NW
EPT16 = EPT // LANES
CH = 64
NCH = EPT // CH

_MESH = plsc.VectorSubcoreMesh(
    core_axis_name="c", subcore_axis_name="s", num_cores=NC, num_subcores=NS
)


def _unpack_src(v):
    return lax.bitwise_and(v, 0xFFFF)


def _unpack_dst(v):
    return lax.shift_right_logical(v, 16)


HR = N // LANES
HCH = 125
NHCH = HR // HCH


def _hist_body(ed_hbm, idrow_hbm, out_hbm, ed_v, idrow_v, hist_v, sem,
               hist_sh):
    c = lax.axis_index("c")
    s = lax.axis_index("s")
    wid = s * NC + c
    cp = pltpu.async_copy(ed_hbm.at[wid], ed_v, sem)
    pltpu.sync_copy(idrow_hbm, idrow_v)

    zeros16 = jnp.zeros((LANES,), jnp.float32)

    def zb(i, carry):
        hist_v[i] = zeros16
        return carry

    lax.fori_loop(0, HR, zb, 0)

    @pl.when(s == 0)
    def _():
        pltpu.sync_copy(hist_v, hist_sh)

    plsc.subcore_barrier()
    cp.wait()

    ones16 = jnp.ones((LANES,), jnp.float32)

    def sb(i, carry):
        idx = _unpack_dst(ed_v[i])
        plsc.addupdate_scatter(
            hist_v, [lax.shift_right_logical(idx, 4), lax.bitwise_and(idx, 15)],
            ones16)
        return carry

    lax.fori_loop(0, EPT16, sb, 0)

    for k in range(NHCH):
        pltpu.sync_copy(hist_v.at[pl.ds(k * HCH, HCH)],
                        hist_sh.at[idrow_v.at[k]], add=True)
    plsc.subcore_barrier()

    @pl.when(s == 0)
    def _():
        pltpu.sync_copy(hist_sh, out_hbm.at[c])


_hist = pl.kernel(
    _hist_body,
    out_type=jax.ShapeDtypeStruct((NC, HR, LANES), jnp.float32),
    mesh=_MESH,
    compiler_params=pltpu.CompilerParams(
        needs_layout_passes=False, use_tc_tiling_on_sc=False),
    scratch_types=[
        pltpu.VMEM((EPT16, LANES), jnp.int32),
        pltpu.VMEM((NHCH, HCH), jnp.int32),
        pltpu.VMEM((HR, LANES), jnp.float32),
        pltpu.SemaphoreType.DMA,
        pltpu.VMEM_SHARED((HR, LANES), jnp.float32),
    ],
    name="gcn_degree_hist",
)


NSPC = 5120
TRASH = 128
ACC_R = NSPC + TRASH
STR_Z = ACC_R // NS
STR_W = NSPC // NS
EPC = E // NS
EDR = 160
EPC_PAD = EDR * 128
NCHP = EPC_PAD // CH
GR = CH // LANES


NBUF = 8


def _agg_body(table_hbm, ed_hbm, out_hbm,
              edidx_v, ed_v, src_vs, dst_vs, rows, sems, sem_ed, acc_sh):
    c = lax.axis_index("c")
    s = lax.axis_index("s")

    iota16 = lax.iota(jnp.int32, LANES)
    base = s * EDR

    def ib(j, carry):
        edidx_v[pl.ds(j * LANES, LANES)] = base + j * LANES + iota16
        return carry

    lax.fori_loop(0, EDR // LANES, ib, 0)
    cp_ed = pltpu.async_copy(ed_hbm.at[edidx_v], ed_v, sem_ed)

    zeros16 = jnp.zeros((LANES,), jnp.float32)
    ncol = D // LANES

    def zb(i, carry):
        rows[0][i // ncol, pl.ds((i % ncol) * LANES, LANES)] = zeros16
        return carry

    lax.fori_loop(0, CH * ncol, zb, 0)

    nz = STR_Z // CH
    for k in range(nz):
        pltpu.sync_copy(rows[0], acc_sh.at[pl.ds(s * STR_Z + k * CH, CH)])
    ztail = STR_Z - nz * CH
    if ztail:
        pltpu.sync_copy(rows[0].at[pl.ds(0, ztail)],
                        acc_sh.at[pl.ds(s * STR_Z + nz * CH, ztail)])

    cp_ed.wait()
    plsc.subcore_barrier()

    cbase = c * NSPC

    def unpack(j, sv, dv):
        for g in range(GR):
            v = lax.bitcast_convert_type(
                ed_v[j, pl.ds(g * LANES, LANES)], jnp.int32)
            src = _unpack_src(v)
            dst = _unpack_dst(v) - cbase
            ok = (dst >= 0) & (dst < NSPC)
            tr = NSPC + ((j * GR + g) & 7) * LANES + iota16
            sv[pl.ds(g * LANES, LANES)] = src
            dv[pl.ds(g * LANES, LANES)] = jnp.where(ok, dst, tr)

    for t in range(NBUF):
        unpack(t, src_vs[t], dst_vs[t])
        pltpu.async_copy(table_hbm.at[src_vs[t]], rows[t], sems[t])

    def ring(m, carry):
        for t in range(NBUF):
            j = m * NBUF + t
            pltpu.make_async_copy(table_hbm.at[src_vs[t]], rows[t],
                                  sems[t]).wait()
            pltpu.sync_copy(rows[t], acc_sh.at[dst_vs[t]], add=True)

            @pl.when(j + NBUF < NCHP)
            def _():
                unpack(j + NBUF, src_vs[t], dst_vs[t])
                pltpu.async_copy(table_hbm.at[src_vs[t]], rows[t], sems[t])
        return carry

    lax.fori_loop(0, NCHP // NBUF, ring, 0)

    plsc.subcore_barrier()
    nw = STR_W // CH
    for k in range(nw):
        pltpu.sync_copy(acc_sh.at[pl.ds(s * STR_W + k * CH, CH)],
                        out_hbm.at[pl.ds(cbase + s * STR_W + k * CH, CH)])
    wtail = STR_W - nw * CH
    if wtail:
        pltpu.sync_copy(
            acc_sh.at[pl.ds(s * STR_W + nw * CH, wtail)],
            out_hbm.at[pl.ds(cbase + s * STR_W + nw * CH, wtail)])


_agg = pl.kernel(
    _agg_body,
    out_type=jax.ShapeDtypeStruct((2 * NSPC, D), jnp.float32),
    mesh=_MESH,
    scratch_types=[
        pltpu.VMEM((EDR,), jnp.int32),
        pltpu.VMEM((EDR, 128), jnp.float32),
        [pltpu.VMEM((CH,), jnp.int32)] * NBUF,
        [pltpu.VMEM((CH,), jnp.int32)] * NBUF,
        [pltpu.VMEM((CH, D), jnp.float32)] * NBUF,
        [pltpu.SemaphoreType.DMA] * NBUF,
        pltpu.SemaphoreType.DMA,
        pltpu.VMEM_SHARED((ACC_R, D), jnp.float32),
    ],
    name="gcn_edge_agg",
)


BR = 1000


def _dot_wt(h, w):
    return lax.dot_general(h, w, (((1,), (1,)), ((), ())),
                           preferred_element_type=jnp.float32)


def _tc0_body(x_ref, w_ref, d0_ref, d1_ref, hw_ref, dis_ref):
    deg = d0_ref[...] + d1_ref[...] + 1.0
    dis = lax.rsqrt(deg)
    hw_ref[...] = _dot_wt(x_ref[...], w_ref[...]) * dis
    dis_ref[...] = dis


def _layernorm_blk(h, g, be):
    mu = jnp.mean(h, axis=-1, keepdims=True)
    hc = h - mu
    var = jnp.mean(hc * hc, axis=-1, keepdims=True)
    return hc * lax.rsqrt(var + 1e-5) * g + be


def _tc_mid_body(p_ref, hwp_ref, dis_ref, b_ref, g_ref, be_ref,
                 res_ref, w_ref, hwn_ref, resn_ref):
    dis = dis_ref[...]
    h = (p_ref[...] + hwp_ref[...]) * dis + b_ref[...]
    hn = _layernorm_blk(h, g_ref[...], be_ref[...])
    r = jnp.maximum(hn, 0.0) + res_ref[...]
    resn_ref[...] = r
    hwn_ref[...] = _dot_wt(r, w_ref[...]) * dis


def _tc_final_body(p_ref, hwp_ref, dis_ref, b_ref, g_ref, be_ref, out_ref):
    h = (p_ref[...] + hwp_ref[...]) * dis_ref[...] + b_ref[...]
    out_ref[...] = _layernorm_blk(h, g_ref[...], be_ref[...])


_row_spec = pl.BlockSpec((BR, D), lambda i: (i, 0))
_col_spec = pl.BlockSpec((BR, 1), lambda i: (i, 0))
_w_spec = pl.BlockSpec((D, D), lambda i: (0, 0))
_vec_spec = pl.BlockSpec((1, D), lambda i: (0, 0))
_GRID = (N // BR,)
_TC_PARAMS = pltpu.CompilerParams(dimension_semantics=("parallel",))

_tc0 = pl.pallas_call(
    _tc0_body,
    grid=_GRID,
    in_specs=[_row_spec, _w_spec, _col_spec, _col_spec],
    out_specs=(_row_spec, _col_spec),
    out_shape=(
        jax.ShapeDtypeStruct((N, D), jnp.float32),
        jax.ShapeDtypeStruct((N, 1), jnp.float32),
    ),
    compiler_params=_TC_PARAMS,
    name="gcn_tc0",
)

_tc_mid = pl.pallas_call(
    _tc_mid_body,
    grid=_GRID,
    in_specs=[_row_spec, _row_spec, _col_spec,
              _vec_spec, _vec_spec, _vec_spec, _row_spec, _w_spec],
    out_specs=(_row_spec, _row_spec),
    out_shape=(
        jax.ShapeDtypeStruct((N, D), jnp.float32),
        jax.ShapeDtypeStruct((N, D), jnp.float32),
    ),
    compiler_params=_TC_PARAMS,
    name="gcn_tc_mid",
)

_tc_final = pl.pallas_call(
    _tc_final_body,
    grid=_GRID,
    in_specs=[_row_spec, _row_spec, _col_spec,
              _vec_spec, _vec_spec, _vec_spec],
    out_specs=_row_spec,
    out_shape=jax.ShapeDtypeStruct((N, D), jnp.float32),
    compiler_params=_TC_PARAMS,
    name="gcn_tc_final",
)


def kernel(x, edge_index, W0, b0, g0, be0, W1, b1, g1, be1, W2, b2, g2, be2,
           W3, b3, g3, be3, W4, b4, g4, be4):
    ed = (edge_index[0] | (edge_index[1] << 16)).reshape(NW, EPT16, LANES)
    ed_words16 = (edge_index[0] | (edge_index[1] << 16)).reshape(NS, EPC)
    padk = jnp.arange(EPC_PAD - EPC, dtype=jnp.int32)
    pad_words = jnp.broadcast_to((padk % 128) | ((2 * NSPC) << 16),
                                 (NS, EPC_PAD - EPC))
    ed_rows_f32 = lax.bitcast_convert_type(
        jnp.concatenate([ed_words16, pad_words], axis=1).reshape(NS * EDR, 128),
        jnp.float32)
    idrow = jnp.arange(HR, dtype=jnp.int32).reshape(NHCH, HCH)

    degp = _hist(ed, idrow)
    d0 = degp[0].reshape(N, 1)
    d1 = degp[1].reshape(N, 1)

    hwp, dis = _tc0(x, W0, d0, d1)

    res = x
    layers = [(b0, g0, be0, W1), (b1, g1, be1, W2),
              (b2, g2, be2, W3), (b3, g3, be3, W4)]
    for (b, g, be, Wn) in layers:
        p = _agg(hwp, ed_rows_f32)
        hwp, res = _tc_mid(p, hwp, dis, b.reshape(1, D), g.reshape(1, D),
                           be.reshape(1, D), res, Wn)
    p = _agg(hwp, ed_rows_f32)
    return _tc_final(p, hwp, dis, b4.reshape(1, D), g4.reshape(1, D),
                     be4.reshape(1, D))


# speedup vs baseline: 22.9729x; 22.9729x over previous
import jax
import jax.numpy as jnp
from jax import lax
from jax.experimental import pallas as pl
from jax.experimental.pallas import tpu as pltpu
from jax.experimental.pallas import tpu_sc as plsc

NC = 2
NS = 16
NW = NC * NS
LANES = 16

N = 10000
D = 128
E = 320000

N_PAD = 10240
SR = N_PAD // NS
EPT = E // NW
EPT16 = EPT // LANES
CH = 128
NCH = EPT // CH

_MESH = plsc.VectorSubcoreMesh(
    core_axis_name="c", subcore_axis_name="s", num_cores=NC, num_subcores=NS
)


def _unpack_src(v):
    return lax.bitwise_and(v, 0xFFFF)


def _unpack_dst(v):
    return lax.shift_right_logical(v, 16)


HR = N // LANES
HCH = 125
NHCH = HR // HCH


def _hist_body(ed_hbm, idrow_hbm, out_hbm, ed_v, idrow_v, hist_v, sem,
               hist_sh):
    c = lax.axis_index("c")
    s = lax.axis_index("s")
    wid = s * NC + c
    cp = pltpu.async_copy(ed_hbm.at[wid], ed_v, sem)
    pltpu.sync_copy(idrow_hbm, idrow_v)

    zeros16 = jnp.zeros((LANES,), jnp.float32)

    def zb(i, carry):
        hist_v[i] = zeros16
        return carry

    lax.fori_loop(0, HR, zb, 0)

    @pl.when(s == 0)
    def _():
        pltpu.sync_copy(hist_v, hist_sh)

    plsc.subcore_barrier()
    cp.wait()

    ones16 = jnp.ones((LANES,), jnp.float32)

    def sb(i, carry):
        idx = _unpack_dst(ed_v[i])
        plsc.addupdate_scatter(
            hist_v, [lax.shift_right_logical(idx, 4), lax.bitwise_and(idx, 15)],
            ones16)
        return carry

    lax.fori_loop(0, EPT16, sb, 0)

    for k in range(NHCH):
        pltpu.sync_copy(hist_v.at[pl.ds(k * HCH, HCH)],
                        hist_sh.at[idrow_v.at[k]], add=True)
    plsc.subcore_barrier()

    @pl.when(s == 0)
    def _():
        pltpu.sync_copy(hist_sh, out_hbm.at[c])


_hist = pl.kernel(
    _hist_body,
    out_type=jax.ShapeDtypeStruct((NC, HR, LANES), jnp.float32),
    mesh=_MESH,
    compiler_params=pltpu.CompilerParams(
        needs_layout_passes=False, use_tc_tiling_on_sc=False),
    scratch_types=[
        pltpu.VMEM((EPT16, LANES), jnp.int32),
        pltpu.VMEM((NHCH, HCH), jnp.int32),
        pltpu.VMEM((HR, LANES), jnp.float32),
        pltpu.SemaphoreType.DMA,
        pltpu.VMEM_SHARED((HR, LANES), jnp.float32),
    ],
    name="gcn_degree_hist",
)


NSPC = 5120
TRASH = 256
ACC_R = NSPC + TRASH
STR_Z = ACC_R // NS
STR_W = NSPC // NS
EPC = E // NS
EDR = 160
EPC_PAD = EDR * 128
NCHP = EPC_PAD // CH
GR = CH // LANES


NBUF = 4


def _agg_body(table_hbm, ed_hbm, out_hbm,
              edidx_v, ed_v, src_vs, dst_vs, rows, sems, sems_s, sem_ed,
              acc_sh):
    c = lax.axis_index("c")
    s = lax.axis_index("s")

    iota16 = lax.iota(jnp.int32, LANES)
    base = s * EDR

    def ib(j, carry):
        edidx_v[pl.ds(j * LANES, LANES)] = base + j * LANES + iota16
        return carry

    lax.fori_loop(0, EDR // LANES, ib, 0)
    cp_ed = pltpu.async_copy(ed_hbm.at[edidx_v], ed_v, sem_ed)

    zeros16 = jnp.zeros((LANES,), jnp.float32)
    ncol = D // LANES

    def zb(i, carry):
        rows[0][i // ncol, pl.ds((i % ncol) * LANES, LANES)] = zeros16
        return carry

    lax.fori_loop(0, CH * ncol, zb, 0)

    nz = STR_Z // CH
    for k in range(nz):
        pltpu.sync_copy(rows[0], acc_sh.at[pl.ds(s * STR_Z + k * CH, CH)])
    ztail = STR_Z - nz * CH
    if ztail:
        pltpu.sync_copy(rows[0].at[pl.ds(0, ztail)],
                        acc_sh.at[pl.ds(s * STR_Z + nz * CH, ztail)])

    cp_ed.wait()
    plsc.subcore_barrier()

    cbase = c * NSPC

    def unpack(j, sv, dv):
        for g in range(GR):
            v = lax.bitcast_convert_type(
                ed_v[j, pl.ds(g * LANES, LANES)], jnp.int32)
            src = _unpack_src(v)
            dst = _unpack_dst(v) - cbase
            ok = (dst >= 0) & (dst < NSPC)
            tr = NSPC + ((j * GR + g) & 15) * LANES + iota16
            sv[pl.ds(g * LANES, LANES)] = src
            dv[pl.ds(g * LANES, LANES)] = jnp.where(ok, dst, tr)

    for t in range(NBUF):
        unpack(t, src_vs[t], dst_vs[t])
        pltpu.async_copy(table_hbm.at[src_vs[t]], rows[t], sems[t])

    def ring(m, carry):
        for t in range(NBUF):
            pltpu.make_async_copy(table_hbm.at[src_vs[t]], rows[t],
                                  sems[t]).wait()
            pltpu.async_copy(rows[t], acc_sh.at[dst_vs[t]], sems_s[t],
                             add=True)
        for t in range(NBUF):
            j = m * NBUF + t
            pltpu.make_async_copy(rows[t], acc_sh.at[dst_vs[t]],
                                  sems_s[t]).wait()

            @pl.when(j + NBUF < NCHP)
            def _():
                unpack(j + NBUF, src_vs[t], dst_vs[t])
                pltpu.async_copy(table_hbm.at[src_vs[t]], rows[t], sems[t])
        return carry

    lax.fori_loop(0, NCHP // NBUF, ring, 0)

    plsc.subcore_barrier()
    nw = STR_W // CH
    for k in range(nw):
        pltpu.sync_copy(acc_sh.at[pl.ds(s * STR_W + k * CH, CH)],
                        out_hbm.at[pl.ds(cbase + s * STR_W + k * CH, CH)])
    wtail = STR_W - nw * CH
    if wtail:
        pltpu.sync_copy(
            acc_sh.at[pl.ds(s * STR_W + nw * CH, wtail)],
            out_hbm.at[pl.ds(cbase + s * STR_W + nw * CH, wtail)])


_agg = pl.kernel(
    _agg_body,
    out_type=jax.ShapeDtypeStruct((2 * NSPC, D), jnp.float32),
    mesh=_MESH,
    scratch_types=[
        pltpu.VMEM((EDR,), jnp.int32),
        pltpu.VMEM((EDR, 128), jnp.float32),
        [pltpu.VMEM((CH,), jnp.int32)] * NBUF,
        [pltpu.VMEM((CH,), jnp.int32)] * NBUF,
        [pltpu.VMEM((CH, D), jnp.float32)] * NBUF,
        [pltpu.SemaphoreType.DMA] * NBUF,
        [pltpu.SemaphoreType.DMA] * NBUF,
        pltpu.SemaphoreType.DMA,
        pltpu.VMEM_SHARED((ACC_R, D), jnp.float32),
    ],
    name="gcn_edge_agg",
)


BR = 1000


def _dot_wt(h, w):
    return lax.dot_general(h, w, (((1,), (1,)), ((), ())),
                           preferred_element_type=jnp.float32)


def _tc0_body(x_ref, w_ref, d0_ref, d1_ref, hw_ref, dis_ref):
    deg = d0_ref[...] + d1_ref[...] + 1.0
    dis = lax.rsqrt(deg)
    hw_ref[...] = _dot_wt(x_ref[...], w_ref[...]) * dis
    dis_ref[...] = dis


def _layernorm_blk(h, g, be):
    mu = jnp.mean(h, axis=-1, keepdims=True)
    hc = h - mu
    var = jnp.mean(hc * hc, axis=-1, keepdims=True)
    return hc * lax.rsqrt(var + 1e-5) * g + be


def _tc_mid_body(p_ref, hwp_ref, dis_ref, b_ref, g_ref, be_ref,
                 res_ref, w_ref, hwn_ref, resn_ref):
    dis = dis_ref[...]
    h = (p_ref[...] + hwp_ref[...]) * dis + b_ref[...]
    hn = _layernorm_blk(h, g_ref[...], be_ref[...])
    r = jnp.maximum(hn, 0.0) + res_ref[...]
    resn_ref[...] = r
    hwn_ref[...] = _dot_wt(r, w_ref[...]) * dis


def _tc_final_body(p_ref, hwp_ref, dis_ref, b_ref, g_ref, be_ref, out_ref):
    h = (p_ref[...] + hwp_ref[...]) * dis_ref[...] + b_ref[...]
    out_ref[...] = _layernorm_blk(h, g_ref[...], be_ref[...])


_row_spec = pl.BlockSpec((BR, D), lambda i: (i, 0))
_col_spec = pl.BlockSpec((BR, 1), lambda i: (i, 0))
_w_spec = pl.BlockSpec((D, D), lambda i: (0, 0))
_vec_spec = pl.BlockSpec((1, D), lambda i: (0, 0))
_GRID = (N // BR,)
_TC_PARAMS = pltpu.CompilerParams(dimension_semantics=("parallel",))

_tc0 = pl.pallas_call(
    _tc0_body,
    grid=_GRID,
    in_specs=[_row_spec, _w_spec, _col_spec, _col_spec],
    out_specs=(_row_spec, _col_spec),
    out_shape=(
        jax.ShapeDtypeStruct((N, D), jnp.float32),
        jax.ShapeDtypeStruct((N, 1), jnp.float32),
    ),
    compiler_params=_TC_PARAMS,
    name="gcn_tc0",
)

_tc_mid = pl.pallas_call(
    _tc_mid_body,
    grid=_GRID,
    in_specs=[_row_spec, _row_spec, _col_spec,
              _vec_spec, _vec_spec, _vec_spec, _row_spec, _w_spec],
    out_specs=(_row_spec, _row_spec),
    out_shape=(
        jax.ShapeDtypeStruct((N, D), jnp.float32),
        jax.ShapeDtypeStruct((N, D), jnp.float32),
    ),
    compiler_params=_TC_PARAMS,
    name="gcn_tc_mid",
)

_tc_final = pl.pallas_call(
    _tc_final_body,
    grid=_GRID,
    in_specs=[_row_spec, _row_spec, _col_spec,
              _vec_spec, _vec_spec, _vec_spec],
    out_specs=_row_spec,
    out_shape=jax.ShapeDtypeStruct((N, D), jnp.float32),
    compiler_params=_TC_PARAMS,
    name="gcn_tc_final",
)


def kernel(x, edge_index, W0, b0, g0, be0, W1, b1, g1, be1, W2, b2, g2, be2,
           W3, b3, g3, be3, W4, b4, g4, be4):
    ed = (edge_index[0] | (edge_index[1] << 16)).reshape(NW, EPT16, LANES)
    ed_words16 = (edge_index[0] | (edge_index[1] << 16)).reshape(NS, EPC)
    padk = jnp.arange(EPC_PAD - EPC, dtype=jnp.int32)
    pad_words = jnp.broadcast_to((padk % 128) | ((2 * NSPC) << 16),
                                 (NS, EPC_PAD - EPC))
    ed_rows_f32 = lax.bitcast_convert_type(
        jnp.concatenate([ed_words16, pad_words], axis=1).reshape(NS * EDR, 128),
        jnp.float32)
    idrow = jnp.arange(HR, dtype=jnp.int32).reshape(NHCH, HCH)

    degp = _hist(ed, idrow)
    d0 = degp[0].reshape(N, 1)
    d1 = degp[1].reshape(N, 1)

    hwp, dis = _tc0(x, W0, d0, d1)

    res = x
    layers = [(b0, g0, be0, W1), (b1, g1, be1, W2),
              (b2, g2, be2, W3), (b3, g3, be3, W4)]
    for (b, g, be, Wn) in layers:
        p = _agg(hwp, ed_rows_f32)
        hwp, res = _tc_mid(p, hwp, dis, b.reshape(1, D), g.reshape(1, D),
                           be.reshape(1, D), res, Wn)
    p = _agg(hwp, ed_rows_f32)
    return _tc_final(p, hwp, dis, b4.reshape(1, D), g4.reshape(1, D),
                     be4.reshape(1, D))


# speedup vs baseline: 26.3333x; 1.1463x over previous
import jax
import jax.numpy as jnp
from jax import lax
from jax.experimental import pallas as pl
from jax.experimental.pallas import tpu as pltpu
from jax.experimental.pallas import tpu_sc as plsc

NC = 2
NS = 16
NW = NC * NS
LANES = 16

N = 10000
D = 128
E = 320000

N_PAD = 10240
SR = N_PAD // NS
EPT = E // NW
EPT16 = EPT // LANES
CH = 128
NCH = EPT // CH

_MESH = plsc.VectorSubcoreMesh(
    core_axis_name="c", subcore_axis_name="s", num_cores=NC, num_subcores=NS
)


def _unpack_src(v):
    return lax.bitwise_and(v, 0xFFFF)


def _unpack_dst(v):
    return lax.shift_right_logical(v, 16)


HR = N // LANES
HCH = 125
NHCH = HR // HCH


def _hist_body(ed_hbm, idrow_hbm, out_hbm, ed_v, idrow_v, hist_v, sem,
               hist_sh):
    c = lax.axis_index("c")
    s = lax.axis_index("s")
    wid = s * NC + c
    cp = pltpu.async_copy(ed_hbm.at[wid], ed_v, sem)
    pltpu.sync_copy(idrow_hbm, idrow_v)

    zeros16 = jnp.zeros((LANES,), jnp.float32)

    def zb(i, carry):
        hist_v[i] = zeros16
        return carry

    lax.fori_loop(0, HR, zb, 0)

    @pl.when(s == 0)
    def _():
        pltpu.sync_copy(hist_v, hist_sh)

    plsc.subcore_barrier()
    cp.wait()

    ones16 = jnp.ones((LANES,), jnp.float32)

    def sb(i, carry):
        idx = _unpack_dst(ed_v[i])
        plsc.addupdate_scatter(
            hist_v, [lax.shift_right_logical(idx, 4), lax.bitwise_and(idx, 15)],
            ones16)
        return carry

    lax.fori_loop(0, EPT16, sb, 0)

    for k in range(NHCH):
        pltpu.sync_copy(hist_v.at[pl.ds(k * HCH, HCH)],
                        hist_sh.at[idrow_v.at[k]], add=True)
    plsc.subcore_barrier()

    @pl.when(s == 0)
    def _():
        pltpu.sync_copy(hist_sh, out_hbm.at[c])


_hist = pl.kernel(
    _hist_body,
    out_type=jax.ShapeDtypeStruct((NC, HR, LANES), jnp.float32),
    mesh=_MESH,
    compiler_params=pltpu.CompilerParams(
        needs_layout_passes=False, use_tc_tiling_on_sc=False),
    scratch_types=[
        pltpu.VMEM((EPT16, LANES), jnp.int32),
        pltpu.VMEM((NHCH, HCH), jnp.int32),
        pltpu.VMEM((HR, LANES), jnp.float32),
        pltpu.SemaphoreType.DMA,
        pltpu.VMEM_SHARED((HR, LANES), jnp.float32),
    ],
    name="gcn_degree_hist",
)


NSPC = 5120
TRASH = 256
ACC_R = NSPC + TRASH
STR_Z = ACC_R // NS
STR_W = NSPC // NS
EPC = E // NS
EDR = 160
EPC_PAD = EDR * 128
NCHP = EPC_PAD // CH
GR = CH // LANES


NBUF = 4


def _agg_body(table_hbm, ed_hbm, out_hbm,
              edidx_v, ed_v, src_vs, dst_vs, rows, sems, sem_ed, acc_sh):
    c = lax.axis_index("c")
    s = lax.axis_index("s")

    iota16 = lax.iota(jnp.int32, LANES)
    base = s * EDR

    def ib(j, carry):
        edidx_v[pl.ds(j * LANES, LANES)] = base + j * LANES + iota16
        return carry

    lax.fori_loop(0, EDR // LANES, ib, 0)
    cp_ed = pltpu.async_copy(ed_hbm.at[edidx_v], ed_v, sem_ed)

    zeros16 = jnp.zeros((LANES,), jnp.float32)
    ncol = D // LANES

    def zb(i, carry):
        rows[0][i // ncol, pl.ds((i % ncol) * LANES, LANES)] = zeros16
        return carry

    lax.fori_loop(0, CH * ncol, zb, 0)

    nz = STR_Z // CH
    for k in range(nz):
        pltpu.sync_copy(rows[0], acc_sh.at[pl.ds(s * STR_Z + k * CH, CH)])
    ztail = STR_Z - nz * CH
    if ztail:
        pltpu.sync_copy(rows[0].at[pl.ds(0, ztail)],
                        acc_sh.at[pl.ds(s * STR_Z + nz * CH, ztail)])

    cp_ed.wait()
    plsc.subcore_barrier()

    cbase = c * NSPC

    def unpack(j, sv, dv):
        for g in range(GR):
            v = lax.bitcast_convert_type(
                ed_v[j, pl.ds(g * LANES, LANES)], jnp.int32)
            src = _unpack_src(v)
            dst = _unpack_dst(v) - cbase
            ok = (dst >= 0) & (dst < NSPC)
            tr = NSPC + ((j * GR + g) & 15) * LANES + iota16
            sv[pl.ds(g * LANES, LANES)] = src
            dv[pl.ds(g * LANES, LANES)] = jnp.where(ok, dst, tr)

    for t in range(NBUF):
        unpack(t, src_vs[t], dst_vs[t])
        pltpu.async_copy(table_hbm.at[src_vs[t]], rows[t], sems[t])

    def ring(m, carry):
        for t in range(NBUF):
            j = m * NBUF + t
            pltpu.make_async_copy(table_hbm.at[src_vs[t]], rows[t],
                                  sems[t]).wait()
            pltpu.sync_copy(rows[t], acc_sh.at[dst_vs[t]], add=True)

            @pl.when(j + NBUF < NCHP)
            def _():
                unpack(j + NBUF, src_vs[t], dst_vs[t])
                pltpu.async_copy(table_hbm.at[src_vs[t]], rows[t], sems[t])
        return carry

    lax.fori_loop(0, NCHP // NBUF, ring, 0)

    plsc.subcore_barrier()
    nw = STR_W // CH
    for k in range(nw):
        pltpu.sync_copy(acc_sh.at[pl.ds(s * STR_W + k * CH, CH)],
                        out_hbm.at[pl.ds(cbase + s * STR_W + k * CH, CH)])
    wtail = STR_W - nw * CH
    if wtail:
        pltpu.sync_copy(
            acc_sh.at[pl.ds(s * STR_W + nw * CH, wtail)],
            out_hbm.at[pl.ds(cbase + s * STR_W + nw * CH, wtail)])


_agg = pl.kernel(
    _agg_body,
    out_type=jax.ShapeDtypeStruct((2 * NSPC, D), jnp.float32),
    mesh=_MESH,
    scratch_types=[
        pltpu.VMEM((EDR,), jnp.int32),
        pltpu.VMEM((EDR, 128), jnp.float32),
        [pltpu.VMEM((CH,), jnp.int32)] * NBUF,
        [pltpu.VMEM((CH,), jnp.int32)] * NBUF,
        [pltpu.VMEM((CH, D), jnp.float32)] * NBUF,
        [pltpu.SemaphoreType.DMA] * NBUF,
        pltpu.SemaphoreType.DMA,
        pltpu.VMEM_SHARED((ACC_R, D), jnp.float32),
    ],
    name="gcn_edge_agg",
)


BR = 1000


def _dot_wt(h, w):
    return lax.dot_general(h, w, (((1,), (1,)), ((), ())),
                           preferred_element_type=jnp.float32)


def _tc0_body(x_ref, w_ref, d0_ref, d1_ref, hw_ref, dis_ref):
    deg = d0_ref[...] + d1_ref[...] + 1.0
    dis = lax.rsqrt(deg)
    hw_ref[...] = _dot_wt(x_ref[...], w_ref[...]) * dis
    dis_ref[...] = dis


def _layernorm_blk(h, g, be):
    mu = jnp.mean(h, axis=-1, keepdims=True)
    hc = h - mu
    var = jnp.mean(hc * hc, axis=-1, keepdims=True)
    return hc * lax.rsqrt(var + 1e-5) * g + be


def _tc_mid_body(p_ref, hwp_ref, dis_ref, b_ref, g_ref, be_ref,
                 res_ref, w_ref, hwn_ref, resn_ref):
    dis = dis_ref[...]
    h = (p_ref[...] + hwp_ref[...]) * dis + b_ref[...]
    hn = _layernorm_blk(h, g_ref[...], be_ref[...])
    r = jnp.maximum(hn, 0.0) + res_ref[...]
    resn_ref[...] = r
    hwn_ref[...] = _dot_wt(r, w_ref[...]) * dis


def _tc_final_body(p_ref, hwp_ref, dis_ref, b_ref, g_ref, be_ref, out_ref):
    h = (p_ref[...] + hwp_ref[...]) * dis_ref[...] + b_ref[...]
    out_ref[...] = _layernorm_blk(h, g_ref[...], be_ref[...])


_row_spec = pl.BlockSpec((BR, D), lambda i: (i, 0))
_col_spec = pl.BlockSpec((BR, 1), lambda i: (i, 0))
_w_spec = pl.BlockSpec((D, D), lambda i: (0, 0))
_vec_spec = pl.BlockSpec((1, D), lambda i: (0, 0))
_GRID = (N // BR,)
_TC_PARAMS = pltpu.CompilerParams(dimension_semantics=("parallel",))

_tc0 = pl.pallas_call(
    _tc0_body,
    grid=_GRID,
    in_specs=[_row_spec, _w_spec, _col_spec, _col_spec],
    out_specs=(_row_spec, _col_spec),
    out_shape=(
        jax.ShapeDtypeStruct((N, D), jnp.float32),
        jax.ShapeDtypeStruct((N, 1), jnp.float32),
    ),
    compiler_params=_TC_PARAMS,
    name="gcn_tc0",
)

_tc_mid = pl.pallas_call(
    _tc_mid_body,
    grid=_GRID,
    in_specs=[_row_spec, _row_spec, _col_spec,
              _vec_spec, _vec_spec, _vec_spec, _row_spec, _w_spec],
    out_specs=(_row_spec, _row_spec),
    out_shape=(
        jax.ShapeDtypeStruct((N, D), jnp.float32),
        jax.ShapeDtypeStruct((N, D), jnp.float32),
    ),
    compiler_params=_TC_PARAMS,
    name="gcn_tc_mid",
)

_tc_final = pl.pallas_call(
    _tc_final_body,
    grid=_GRID,
    in_specs=[_row_spec, _row_spec, _col_spec,
              _vec_spec, _vec_spec, _vec_spec],
    out_specs=_row_spec,
    out_shape=jax.ShapeDtypeStruct((N, D), jnp.float32),
    compiler_params=_TC_PARAMS,
    name="gcn_tc_final",
)


def kernel(x, edge_index, W0, b0, g0, be0, W1, b1, g1, be1, W2, b2, g2, be2,
           W3, b3, g3, be3, W4, b4, g4, be4):
    ed = (edge_index[0] | (edge_index[1] << 16)).reshape(NW, EPT16, LANES)
    ed_words16 = (edge_index[0] | (edge_index[1] << 16)).reshape(NS, EPC)
    padk = jnp.arange(EPC_PAD - EPC, dtype=jnp.int32)
    pad_words = jnp.broadcast_to((padk % 128) | ((2 * NSPC) << 16),
                                 (NS, EPC_PAD - EPC))
    ed_rows_f32 = lax.bitcast_convert_type(
        jnp.concatenate([ed_words16, pad_words], axis=1).reshape(NS * EDR, 128),
        jnp.float32)
    idrow = jnp.arange(HR, dtype=jnp.int32).reshape(NHCH, HCH)

    degp = _hist(ed, idrow)
    d0 = degp[0].reshape(N, 1)
    d1 = degp[1].reshape(N, 1)

    hwp, dis = _tc0(x, W0, d0, d1)

    res = x
    layers = [(b0, g0, be0, W1), (b1, g1, be1, W2),
              (b2, g2, be2, W3), (b3, g3, be3, W4)]
    for (b, g, be, Wn) in layers:
        p = _agg(hwp, ed_rows_f32)
        hwp, res = _tc_mid(p, hwp, dis, b.reshape(1, D), g.reshape(1, D),
                           be.reshape(1, D), res, Wn)
    p = _agg(hwp, ed_rows_f32)
    return _tc_final(p, hwp, dis, b4.reshape(1, D), g4.reshape(1, D),
                     be4.reshape(1, D))
